# baseline (speedup 1.0000x reference)
import jax
import jax.numpy as jnp
from jax import lax
from jax.experimental import pallas as pl
from jax.experimental.pallas import tpu as pltpu
from jax.experimental.pallas import tpu_sc as plsc

_NC, _NS = 2, 16
_NW = _NC * _NS
_C, _J, _K = 3, 144, 118
_ROWS = _C * _K
_RPW = -(-_ROWS // _NW)


def _sc_body(x3, maps_hbm, out3, maps_v, bufs, sem_a, sem_b):
    w = lax.axis_index("sub") * _NC + lax.axis_index("core")
    pltpu.sync_copy(maps_hbm, maps_v.at[pl.ds(0, _K)])
    sems = (sem_a, sem_b)

    def row_coords(t):
        r = w + _NW * t
        valid = r < _ROWS
        rc = jnp.where(valid, r, 0)
        ci = rc // _K
        j = rc - ci * _K
        base = (j // 16) * 16
        m_vec = maps_v[pl.ds(base, 16)]
        lane = lax.broadcasted_iota(jnp.int32, (16,), 0)
        m = jnp.max(jnp.where(lane == j - base, m_vec, -1))
        return valid, ci, j, m

    coords = [row_coords(t) for t in range(_RPW)]

    def start_in(t):
        valid, ci, _, m = coords[t]

        @pl.when(valid)
        def _():
            pltpu.async_copy(x3.at[ci, m, :], bufs.at[t % 2], sems[t % 2])

    start_in(0)
    for t in range(_RPW):
        if t + 1 < _RPW:
            start_in(t + 1)
        valid, ci, j, m = coords[t]

        @pl.when(valid)
        def _():
            pltpu.make_async_copy(
                x3.at[ci, m, :], bufs.at[t % 2], sems[t % 2]
            ).wait()
            pltpu.sync_copy(bufs.at[t % 2], out3.at[ci, j, :])


def _sc_gather(x3, maps):
    mesh = plsc.VectorSubcoreMesh(
        core_axis_name="core", subcore_axis_name="sub",
        num_cores=_NC, num_subcores=_NS,
    )
    return pl.kernel(
        _sc_body,
        out_type=jax.ShapeDtypeStruct((_C, _K, 16384), jnp.float32),
        mesh=mesh,
        compiler_params=pltpu.CompilerParams(
            needs_layout_passes=False, use_tc_tiling_on_sc=True,
        ),
        scratch_types=[
            pltpu.VMEM((128,), jnp.int32),
            pltpu.VMEM((2, 16384), jnp.float32),
            pltpu.SemaphoreType.DMA,
            pltpu.SemaphoreType.DMA,
        ],
    )(x3, maps)


def kernel(joints, joint_maps):
    b, j, c = joints.shape
    x3 = jnp.transpose(joints, (2, 1, 0))
    out3 = _sc_gather(x3, joint_maps)
    return jnp.transpose(out3, (2, 1, 0))


# speedup vs baseline: 1.0312x; 1.0312x over previous
import jax
import jax.numpy as jnp
from jax import lax
from jax.experimental import pallas as pl
from jax.experimental.pallas import tpu as pltpu
from jax.experimental.pallas import tpu_sc as plsc

_NC, _NS = 2, 16
_NW = _NC * _NS
_C, _J, _K = 3, 144, 118
_ROWS = _C * _K
_RPW = -(-_ROWS // _NW)
_SLOTS = 4
_LOOK = 2


def _sc_body(x3, maps_hbm, out3, maps_v, bufs, *sems):
    w = lax.axis_index("sub") * _NC + lax.axis_index("core")
    pltpu.sync_copy(maps_hbm, maps_v.at[pl.ds(0, _K)])
    sems_in, sems_out = sems[:_SLOTS], sems[_SLOTS:]

    def row_coords(t):
        r = w + _NW * t
        valid = r < _ROWS
        rc = jnp.where(valid, r, 0)
        ci = rc // _K
        j = rc - ci * _K
        base = (j // 16) * 16
        m_vec = maps_v[pl.ds(base, 16)]
        lane = lax.broadcasted_iota(jnp.int32, (16,), 0)
        m = jnp.max(jnp.where(lane == j - base, m_vec, -1))
        return valid, ci, j, m

    coords = [row_coords(t) for t in range(_RPW)]

    def in_src(t):
        _, ci, _, m = coords[t]
        return x3.at[ci, m, :]

    def out_dst(t):
        _, ci, j, _ = coords[t]
        return out3.at[ci, j, :]

    def buf(t):
        return bufs.at[t % _SLOTS]

    def fire_in(t):
        @pl.when(coords[t][0])
        def _():
            pltpu.async_copy(in_src(t), buf(t), sems_in[t % _SLOTS])

    def wait_in(t):
        @pl.when(coords[t][0])
        def _():
            pltpu.make_async_copy(in_src(t), buf(t),
                                  sems_in[t % _SLOTS]).wait()

    def fire_out(t):
        @pl.when(coords[t][0])
        def _():
            pltpu.async_copy(buf(t), out_dst(t), sems_out[t % _SLOTS])

    def wait_out(t):
        @pl.when(coords[t][0])
        def _():
            pltpu.make_async_copy(buf(t), out_dst(t),
                                  sems_out[t % _SLOTS]).wait()

    for t in range(min(_LOOK, _RPW)):
        fire_in(t)
    for t in range(_RPW):
        nt = t + _LOOK
        if nt < _RPW:
            if nt - _SLOTS >= 0:
                wait_out(nt - _SLOTS)
            fire_in(nt)
        wait_in(t)
        fire_out(t)
    for t in range(max(0, _RPW - _SLOTS), _RPW):
        wait_out(t)


def _sc_gather(x3, maps):
    mesh = plsc.VectorSubcoreMesh(
        core_axis_name="core", subcore_axis_name="sub",
        num_cores=_NC, num_subcores=_NS,
    )
    return pl.kernel(
        _sc_body,
        out_type=jax.ShapeDtypeStruct((_C, _K, 16384), jnp.float32),
        mesh=mesh,
        compiler_params=pltpu.CompilerParams(
            needs_layout_passes=False, use_tc_tiling_on_sc=True,
        ),
        scratch_types=(
            [pltpu.VMEM((128,), jnp.int32),
             pltpu.VMEM((_SLOTS, 16384), jnp.float32)]
            + [pltpu.SemaphoreType.DMA] * (2 * _SLOTS)
        ),
    )(x3, maps)


def kernel(joints, joint_maps):
    b, j, c = joints.shape
    x3 = jnp.transpose(joints, (2, 1, 0))
    out3 = _sc_gather(x3, joint_maps)
    return jnp.transpose(out3, (2, 1, 0))


# speedup vs baseline: 2.3469x; 2.2758x over previous
import jax
import jax.numpy as jnp
from jax.experimental import pallas as pl
from jax.experimental.pallas import tpu as pltpu


def _gather_body(maps_ref, x_ref, o_ref):
    c, j, _ = x_ref.shape
    k = o_ref.shape[1]
    maps = maps_ref[0, :]
    cols = jax.lax.broadcasted_iota(jnp.int32, (k, j), 1)
    sel = jnp.where(cols == maps[:, None], 1.0, 0.0).astype(jnp.float32)
    for ci in range(c):
        o_ref[ci] = jnp.dot(sel, x_ref[ci], preferred_element_type=jnp.float32)


def kernel(joints, joint_maps):
    b, j, c = joints.shape
    k = joint_maps.shape[0]
    xt = jnp.transpose(joints, (2, 1, 0))
    maps = joint_maps.reshape(1, k)
    blk = 8192
    out_t = pl.pallas_call(
        _gather_body,
        grid=(b // blk,),
        in_specs=[
            pl.BlockSpec((1, k), lambda i: (0, 0)),
            pl.BlockSpec((c, j, blk), lambda i: (0, 0, i)),
        ],
        out_specs=pl.BlockSpec((c, k, blk), lambda i: (0, 0, i)),
        out_shape=jax.ShapeDtypeStruct((c, k, b), jnp.float32),
    )(maps, xt)
    return jnp.transpose(out_t, (2, 1, 0))
